# baseline (speedup 1.0000x reference)
import functools

import jax
import jax.numpy as jnp
from jax import lax
from jax.experimental import pallas as pl
from jax.experimental.pallas import tpu as pltpu
from jax.experimental.pallas import tpu_sc as plsc

NC, NS, LANES = 2, 16, 16
NW = NC * NS
CH = 128

N = 10000
NP = 10240
ROWS_PER_SUB = NP // NS
DEG_W = 16
BLK = 1280
W2P = 256

_MESH = plsc.VectorSubcoreMesh(core_axis_name="c", subcore_axis_name="s")
_SC_PARAMS = pltpu.CompilerParams(use_tc_tiling_on_sc=False)


def _zero_vmem(buf, nrows, width):
    z = jnp.zeros((LANES,), jnp.float32)

    @pl.loop(0, nrows)
    def _(r):
        @pl.loop(0, width // LANES)
        def _(c):
            buf[r, pl.ds(c * LANES, LANES)] = z


def _make_sc_degree(n_edges_p):
    chunks_per_w = n_edges_p // (NW * CH)

    @functools.partial(
        pl.kernel,
        out_type=jax.ShapeDtypeStruct((NC, NP, DEG_W), jnp.float32),
        mesh=_MESH,
        compiler_params=_SC_PARAMS,
        scratch_types=[
            pltpu.VMEM((chunks_per_w, CH), jnp.int32),
            pltpu.VMEM((CH, DEG_W), jnp.float32),
            pltpu.VMEM((CH, DEG_W), jnp.float32),
            pltpu.VMEM_SHARED((NP, DEG_W), jnp.float32),
            pltpu.SemaphoreType.DMA,
        ],
    )
    def sc_degree(dst_hbm, out_hbm, idx_v, ones_v, zbuf, acc, sem):
        cid = lax.axis_index("c")
        sid = lax.axis_index("s")
        wid = cid * NS + sid
        base = wid * chunks_per_w
        one = jnp.full((LANES,), 1.0, jnp.float32)

        pltpu.async_copy(dst_hbm.at[pl.ds(base, chunks_per_w)], idx_v, sem)

        @pl.loop(0, CH)
        def _(r):
            ones_v[r, :] = one

        _zero_vmem(zbuf, CH, DEG_W)

        @pl.loop(0, ROWS_PER_SUB // CH)
        def _(j):
            pltpu.sync_copy(zbuf, acc.at[pl.ds(sid * ROWS_PER_SUB + j * CH, CH)])

        pltpu.make_async_copy(
            dst_hbm.at[pl.ds(base, chunks_per_w)], idx_v, sem).wait()
        plsc.subcore_barrier()

        @pl.loop(0, chunks_per_w)
        def _(i):
            pltpu.sync_copy(ones_v, acc.at[idx_v.at[i]], add=True)

        plsc.subcore_barrier()
        r0 = sid * ROWS_PER_SUB
        pltpu.sync_copy(acc.at[pl.ds(r0, ROWS_PER_SUB)],
                        out_hbm.at[cid, pl.ds(r0, ROWS_PER_SUB)])

    return sc_degree


def _make_sc_scatter(n_edges_p):
    chunks_per_w = n_edges_p // (NW * CH)
    assert chunks_per_w % 2 == 0

    @functools.partial(
        pl.kernel,
        out_type=jax.ShapeDtypeStruct((NC, NP, 128), jnp.float32),
        mesh=_MESH,
        compiler_params=_SC_PARAMS,
        scratch_types=[
            pltpu.VMEM((chunks_per_w, CH), jnp.int32),
            pltpu.VMEM((chunks_per_w, CH), jnp.int32),
            pltpu.VMEM((CH, 128), jnp.float32),
            pltpu.VMEM_SHARED((NP, 128), jnp.float32),
            pltpu.SemaphoreType.DMA,
        ],
    )
    def sc_scatter(y_hbm, src_hbm, dst_hbm, out_hbm, srcv, dstv, rows, acc, sem):
        cid = lax.axis_index("c")
        sid = lax.axis_index("s")
        wid = cid * NS + sid
        base = wid * chunks_per_w

        pltpu.async_copy(src_hbm.at[pl.ds(base, chunks_per_w)], srcv, sem)
        pltpu.async_copy(dst_hbm.at[pl.ds(base, chunks_per_w)], dstv, sem)

        _zero_vmem(rows, CH, 128)

        @pl.loop(0, ROWS_PER_SUB // CH)
        def _(j):
            pltpu.sync_copy(rows, acc.at[pl.ds(sid * ROWS_PER_SUB + j * CH, CH)])

        pltpu.make_async_copy(
            src_hbm.at[pl.ds(base, chunks_per_w)], srcv, sem).wait()
        pltpu.make_async_copy(
            dst_hbm.at[pl.ds(base, chunks_per_w)], dstv, sem).wait()
        plsc.subcore_barrier()

        @pl.loop(0, chunks_per_w)
        def _(i):
            pltpu.async_copy(y_hbm.at[srcv.at[i]], rows, sem).wait()
            pltpu.sync_copy(rows, acc.at[dstv.at[i]], add=True)

        plsc.subcore_barrier()
        r0 = sid * ROWS_PER_SUB
        pltpu.sync_copy(acc.at[pl.ds(r0, ROWS_PER_SUB)],
                        out_hbm.at[cid, pl.ds(r0, ROWS_PER_SUB)])

    return sc_scatter


def _dinv_from(deg_ref):
    deg = deg_ref[0, :, 0] + deg_ref[1, :, 0] + 1.0
    return lax.rsqrt(deg)[:, None]


def _tc_scale(x, deg):
    def body(x_ref, deg_ref, o_ref):
        o_ref[...] = x_ref[...] * _dinv_from(deg_ref)

    return pl.pallas_call(
        body,
        grid=(NP // BLK,),
        in_specs=[
            pl.BlockSpec((BLK, 128), lambda i: (i, 0)),
            pl.BlockSpec((NC, BLK, DEG_W), lambda i: (0, i, 0)),
        ],
        out_specs=pl.BlockSpec((BLK, 128), lambda i: (i, 0)),
        out_shape=jax.ShapeDtypeStruct((NP, 128), jnp.float32),
    )(x, deg)


def _tc_mid(Z1, xs, deg, w1, b1):
    def body(z_ref, x_ref, deg_ref, w_ref, b_ref, o_ref):
        dinv = _dinv_from(deg_ref)
        u = dinv * (z_ref[0] + z_ref[1] + x_ref[...])
        h = jnp.dot(u, w_ref[...], preferred_element_type=jnp.float32)
        h = jnp.maximum(h + b_ref[...], 0.0)
        o_ref[...] = dinv * h

    return pl.pallas_call(
        body,
        grid=(NP // BLK,),
        in_specs=[
            pl.BlockSpec((NC, BLK, 128), lambda i: (0, i, 0)),
            pl.BlockSpec((BLK, 128), lambda i: (i, 0)),
            pl.BlockSpec((NC, BLK, DEG_W), lambda i: (0, i, 0)),
            pl.BlockSpec((128, 128), lambda i: (0, 0)),
            pl.BlockSpec((1, 128), lambda i: (0, 0)),
        ],
        out_specs=pl.BlockSpec((BLK, 128), lambda i: (i, 0)),
        out_shape=jax.ShapeDtypeStruct((NP, 128), jnp.float32),
    )(Z1, xs, deg, w1, b1)


def _tc_final(Z2, hs, deg, w2p, b2p):
    def body(z_ref, h_ref, deg_ref, w_ref, b_ref, o_ref):
        dinv = _dinv_from(deg_ref)
        u = dinv * (z_ref[0] + z_ref[1] + h_ref[...])
        o = jnp.dot(u, w_ref[...], preferred_element_type=jnp.float32)
        o = o + b_ref[...]
        col = lax.broadcasted_iota(jnp.int32, o.shape, 1)
        o_ref[...] = jnp.where(col >= 128, jnp.exp(o), o)

    return pl.pallas_call(
        body,
        grid=(NP // BLK,),
        in_specs=[
            pl.BlockSpec((NC, BLK, 128), lambda i: (0, i, 0)),
            pl.BlockSpec((BLK, 128), lambda i: (i, 0)),
            pl.BlockSpec((NC, BLK, DEG_W), lambda i: (0, i, 0)),
            pl.BlockSpec((128, W2P), lambda i: (0, 0)),
            pl.BlockSpec((1, W2P), lambda i: (0, 0)),
        ],
        out_specs=pl.BlockSpec((BLK, W2P), lambda i: (i, 0)),
        out_shape=jax.ShapeDtypeStruct((NP, W2P), jnp.float32),
    )(Z2, hs, deg, w2p, b2p)


def kernel(x, edge_index, W1, b1, W2, b2):
    n_edges = edge_index.shape[1]
    chunk = NW * CH * 2
    n_edges_p = ((n_edges + chunk - 1) // chunk) * chunk

    src_p = jnp.pad(edge_index[0], (0, n_edges_p - n_edges),
                    constant_values=N).reshape(n_edges_p // CH, CH)
    dst_p = jnp.pad(edge_index[1], (0, n_edges_p - n_edges),
                    constant_values=N).reshape(n_edges_p // CH, CH)
    xp = jnp.pad(x, ((0, NP - N), (0, 0)))
    w2p = jnp.pad(W2, ((0, 0), (0, W2P - W2.shape[1])))
    b1r = b1.reshape(1, 128)
    b2p = jnp.pad(b2, (0, W2P - b2.shape[0])).reshape(1, W2P)

    sc_degree = _make_sc_degree(n_edges_p)
    sc_scatter = _make_sc_scatter(n_edges_p)

    deg = sc_degree(dst_p)
    xs = _tc_scale(xp, deg)
    Z1 = sc_scatter(xs, src_p, dst_p)
    hs = _tc_mid(Z1, xs, deg, W1, b1r)
    Z2 = sc_scatter(hs, src_p, dst_p)
    out = _tc_final(Z2, hs, deg, w2p, b2p)

    pos = out[:N, :128]
    mass = out[:N, 128:129]
    return (pos, mass)


# speedup vs baseline: 1.1468x; 1.1468x over previous
import functools

import jax
import jax.numpy as jnp
from jax import lax
from jax.experimental import pallas as pl
from jax.experimental.pallas import tpu as pltpu
from jax.experimental.pallas import tpu_sc as plsc

NC, NS, LANES = 2, 16, 16
NW = NC * NS
CH = 128

N = 10000
NP = 10240
ROWS_PER_SUB = NP // NS
DEG_W = 16
BLK = 1280
W2P = 256

_MESH = plsc.VectorSubcoreMesh(core_axis_name="c", subcore_axis_name="s")
_SC_PARAMS = pltpu.CompilerParams(use_tc_tiling_on_sc=False)


def _zero_vmem(buf, nrows, width):
    z = jnp.zeros((LANES,), jnp.float32)

    @pl.loop(0, nrows)
    def _(r):
        @pl.loop(0, width // LANES)
        def _(c):
            buf[r, pl.ds(c * LANES, LANES)] = z


def _make_sc_degree(n_edges_p):
    chunks_per_w = n_edges_p // (NW * CH)

    @functools.partial(
        pl.kernel,
        out_type=jax.ShapeDtypeStruct((NC, NP, DEG_W), jnp.float32),
        mesh=_MESH,
        compiler_params=_SC_PARAMS,
        scratch_types=[
            pltpu.VMEM((CH,), jnp.int32),
            pltpu.VMEM((CH, DEG_W), jnp.float32),
            pltpu.VMEM((CH, DEG_W), jnp.float32),
            pltpu.VMEM_SHARED((NP, DEG_W), jnp.float32),
        ],
    )
    def sc_degree(dst_hbm, out_hbm, idx_v, ones_v, zbuf, acc):
        cid = lax.axis_index("c")
        sid = lax.axis_index("s")
        wid = cid * NS + sid
        base = wid * chunks_per_w
        one = jnp.full((LANES,), 1.0, jnp.float32)

        @pl.loop(0, CH)
        def _(r):
            ones_v[r, :] = one

        _zero_vmem(zbuf, CH, DEG_W)

        @pl.loop(0, ROWS_PER_SUB // CH)
        def _(j):
            pltpu.sync_copy(zbuf, acc.at[pl.ds(sid * ROWS_PER_SUB + j * CH, CH)])

        plsc.subcore_barrier()

        @pl.loop(0, chunks_per_w)
        def _(i):
            pltpu.sync_copy(dst_hbm.at[base + i], idx_v)
            pltpu.sync_copy(ones_v, acc.at[idx_v], add=True)

        plsc.subcore_barrier()
        r0 = sid * ROWS_PER_SUB
        pltpu.sync_copy(acc.at[pl.ds(r0, ROWS_PER_SUB)],
                        out_hbm.at[cid, pl.ds(r0, ROWS_PER_SUB)])

    return sc_degree


def _make_sc_scatter(n_edges_p, chunks_c0):
    total_chunks = n_edges_p // CH
    chunks_c1 = total_chunks // NS - chunks_c0

    @functools.partial(
        pl.kernel,
        out_type=jax.ShapeDtypeStruct((NC, NP, 128), jnp.float32),
        mesh=_MESH,
        compiler_params=_SC_PARAMS,
        scratch_types=[
            pltpu.VMEM((CH,), jnp.int32),
            pltpu.VMEM((CH,), jnp.int32),
            pltpu.VMEM((CH, 128), jnp.float32),
            pltpu.VMEM_SHARED((NP, 128), jnp.float32),
            pltpu.SemaphoreType.DMA,
        ],
    )
    def sc_scatter(y_hbm, src_hbm, dst_hbm, out_hbm, srcv, dstv, rows, acc, sem):
        cid = lax.axis_index("c")
        sid = lax.axis_index("s")
        nch = jnp.where(cid == 0, chunks_c0, chunks_c1)
        base = cid * NS * chunks_c0 + sid * nch

        _zero_vmem(rows, CH, 128)

        @pl.loop(0, ROWS_PER_SUB // CH)
        def _(j):
            pltpu.sync_copy(rows, acc.at[pl.ds(sid * ROWS_PER_SUB + j * CH, CH)])

        plsc.subcore_barrier()

        @pl.loop(0, nch)
        def _(i):
            pltpu.sync_copy(src_hbm.at[base + i], srcv)
            pltpu.sync_copy(dst_hbm.at[base + i], dstv)
            pltpu.async_copy(y_hbm.at[srcv], rows, sem).wait()
            pltpu.sync_copy(rows, acc.at[dstv], add=True)

        plsc.subcore_barrier()
        r0 = sid * ROWS_PER_SUB
        pltpu.sync_copy(acc.at[pl.ds(r0, ROWS_PER_SUB)],
                        out_hbm.at[cid, pl.ds(r0, ROWS_PER_SUB)])

    return sc_scatter


def _dinv_from(deg_ref):
    deg = deg_ref[0, :, 0] + deg_ref[1, :, 0] + 1.0
    return lax.rsqrt(deg)[:, None]


def _tc_scale(x, deg):
    def body(x_ref, deg_ref, o_ref):
        o_ref[...] = x_ref[...] * _dinv_from(deg_ref)

    return pl.pallas_call(
        body,
        grid=(NP // BLK,),
        in_specs=[
            pl.BlockSpec((BLK, 128), lambda i: (i, 0)),
            pl.BlockSpec((NC, BLK, DEG_W), lambda i: (0, i, 0)),
        ],
        out_specs=pl.BlockSpec((BLK, 128), lambda i: (i, 0)),
        out_shape=jax.ShapeDtypeStruct((NP, 128), jnp.float32),
    )(x, deg)


def _tc_mid(Z1, xs, deg, w1, b1):
    def body(z_ref, x_ref, deg_ref, w_ref, b_ref, o_ref):
        dinv = _dinv_from(deg_ref)
        u = dinv * (z_ref[0] + z_ref[1] + x_ref[...])
        h = jnp.dot(u, w_ref[...], preferred_element_type=jnp.float32)
        h = jnp.maximum(h + b_ref[...], 0.0)
        o_ref[...] = dinv * h

    return pl.pallas_call(
        body,
        grid=(NP // BLK,),
        in_specs=[
            pl.BlockSpec((NC, BLK, 128), lambda i: (0, i, 0)),
            pl.BlockSpec((BLK, 128), lambda i: (i, 0)),
            pl.BlockSpec((NC, BLK, DEG_W), lambda i: (0, i, 0)),
            pl.BlockSpec((128, 128), lambda i: (0, 0)),
            pl.BlockSpec((1, 128), lambda i: (0, 0)),
        ],
        out_specs=pl.BlockSpec((BLK, 128), lambda i: (i, 0)),
        out_shape=jax.ShapeDtypeStruct((NP, 128), jnp.float32),
    )(Z1, xs, deg, w1, b1)


def _tc_final(Z2, hs, deg, w2p, b2p):
    def body(z_ref, h_ref, deg_ref, w_ref, b_ref, o_ref):
        dinv = _dinv_from(deg_ref)
        u = dinv * (z_ref[0] + z_ref[1] + h_ref[...])
        o = jnp.dot(u, w_ref[...], preferred_element_type=jnp.float32)
        o = o + b_ref[...]
        col = lax.broadcasted_iota(jnp.int32, o.shape, 1)
        o_ref[...] = jnp.where(col >= 128, jnp.exp(o), o)

    return pl.pallas_call(
        body,
        grid=(NP // BLK,),
        in_specs=[
            pl.BlockSpec((NC, BLK, 128), lambda i: (0, i, 0)),
            pl.BlockSpec((BLK, 128), lambda i: (i, 0)),
            pl.BlockSpec((NC, BLK, DEG_W), lambda i: (0, i, 0)),
            pl.BlockSpec((128, W2P), lambda i: (0, 0)),
            pl.BlockSpec((1, W2P), lambda i: (0, 0)),
        ],
        out_specs=pl.BlockSpec((BLK, W2P), lambda i: (i, 0)),
        out_shape=jax.ShapeDtypeStruct((NP, W2P), jnp.float32),
    )(Z2, hs, deg, w2p, b2p)


def kernel(x, edge_index, W1, b1, W2, b2):
    n_edges = edge_index.shape[1]
    chunk = NW * CH
    n_edges_p = ((n_edges + chunk - 1) // chunk) * chunk

    src_p = jnp.pad(edge_index[0], (0, n_edges_p - n_edges),
                    constant_values=N).reshape(n_edges_p // CH, CH)
    dst_p = jnp.pad(edge_index[1], (0, n_edges_p - n_edges),
                    constant_values=N).reshape(n_edges_p // CH, CH)
    xp = jnp.pad(x, ((0, NP - N), (0, 0)))
    w2p = jnp.pad(W2, ((0, 0), (0, W2P - W2.shape[1])))
    b1r = b1.reshape(1, 128)
    b2p = jnp.pad(b2, (0, W2P - b2.shape[0])).reshape(1, W2P)

    sc_degree = _make_sc_degree(n_edges_p)
    sc_scatter = _make_sc_scatter(n_edges_p, chunks_c0=61)

    deg = sc_degree(dst_p)
    xs = _tc_scale(xp, deg)
    Z1 = sc_scatter(xs, src_p, dst_p)
    hs = _tc_mid(Z1, xs, deg, W1, b1r)
    Z2 = sc_scatter(hs, src_p, dst_p)
    out = _tc_final(Z2, hs, deg, w2p, b2p)

    pos = out[:N, :128]
    mass = out[:N, 128:129]
    return (pos, mass)


# speedup vs baseline: 1.4075x; 1.2273x over previous
import functools

import jax
import jax.numpy as jnp
from jax import lax
from jax.experimental import pallas as pl
from jax.experimental.pallas import tpu as pltpu
from jax.experimental.pallas import tpu_sc as plsc

NC, NS, LANES = 2, 16, 16
NW = NC * NS
CH = 128

N = 10000
NP = 10240
ROWS_PER_SUB = NP // NS
DEG_W = 16
BLK = 1280
W2P = 256

_MESH = plsc.VectorSubcoreMesh(core_axis_name="c", subcore_axis_name="s")
_SC_PARAMS = pltpu.CompilerParams(use_tc_tiling_on_sc=False)


def _zero_vmem(buf, nrows, width):
    z = jnp.zeros((LANES,), jnp.float32)

    @pl.loop(0, nrows)
    def _(r):
        @pl.loop(0, width // LANES)
        def _(c):
            buf[r, pl.ds(c * LANES, LANES)] = z


def _make_sc_degree(n_edges_p):
    chunks_per_w = n_edges_p // (NW * CH)

    @functools.partial(
        pl.kernel,
        out_type=jax.ShapeDtypeStruct((NC, NP, DEG_W), jnp.float32),
        mesh=_MESH,
        compiler_params=_SC_PARAMS,
        scratch_types=[
            pltpu.VMEM((CH,), jnp.int32),
            pltpu.VMEM((CH, DEG_W), jnp.float32),
            pltpu.VMEM((CH, DEG_W), jnp.float32),
            pltpu.VMEM_SHARED((NP, DEG_W), jnp.float32),
        ],
    )
    def sc_degree(dst_hbm, out_hbm, idx_v, ones_v, zbuf, acc):
        cid = lax.axis_index("c")
        sid = lax.axis_index("s")
        wid = cid * NS + sid
        base = wid * chunks_per_w
        one = jnp.full((LANES,), 1.0, jnp.float32)

        @pl.loop(0, CH)
        def _(r):
            ones_v[r, :] = one

        _zero_vmem(zbuf, CH, DEG_W)

        @pl.loop(0, ROWS_PER_SUB // CH)
        def _(j):
            pltpu.sync_copy(zbuf, acc.at[pl.ds(sid * ROWS_PER_SUB + j * CH, CH)])

        plsc.subcore_barrier()

        @pl.loop(0, chunks_per_w)
        def _(i):
            pltpu.sync_copy(dst_hbm.at[base + i], idx_v)
            pltpu.sync_copy(ones_v, acc.at[idx_v], add=True)

        plsc.subcore_barrier()
        r0 = sid * ROWS_PER_SUB
        pltpu.sync_copy(acc.at[pl.ds(r0, ROWS_PER_SUB)],
                        out_hbm.at[cid, pl.ds(r0, ROWS_PER_SUB)])

    return sc_degree


def _make_sc_scatter(n_edges_p, chunks_c0):
    total_chunks = n_edges_p // CH
    chunks_c1 = total_chunks // NS - chunks_c0

    @functools.partial(
        pl.kernel,
        out_type=jax.ShapeDtypeStruct((NC, NP, 128), jnp.float32),
        mesh=_MESH,
        compiler_params=_SC_PARAMS,
        scratch_types=[
            pltpu.VMEM((CH,), jnp.int32),
            pltpu.VMEM((CH,), jnp.int32),
            pltpu.VMEM((CH, 128), jnp.float32),
            pltpu.VMEM_SHARED((NP, 128), jnp.float32),
            pltpu.SemaphoreType.DMA,
        ],
    )
    def sc_scatter(y_hbm, src_hbm, dst_hbm, out_hbm, srcv, dstv, rows, acc, sem):
        cid = lax.axis_index("c")
        sid = lax.axis_index("s")
        nch = jnp.where(cid == 0, chunks_c0, chunks_c1)
        base = cid * NS * chunks_c0 + sid * nch

        _zero_vmem(rows, CH, 128)

        @pl.loop(0, ROWS_PER_SUB // CH)
        def _(j):
            pltpu.sync_copy(rows, acc.at[pl.ds(sid * ROWS_PER_SUB + j * CH, CH)])

        plsc.subcore_barrier()

        @pl.loop(0, nch)
        def _(i):
            pltpu.sync_copy(src_hbm.at[base + i], srcv)
            pltpu.sync_copy(dst_hbm.at[base + i], dstv)
            pltpu.async_copy(y_hbm.at[srcv], rows, sem).wait()
            pltpu.sync_copy(rows, acc.at[dstv], add=True)

        plsc.subcore_barrier()
        r0 = sid * ROWS_PER_SUB
        pltpu.sync_copy(acc.at[pl.ds(r0, ROWS_PER_SUB)],
                        out_hbm.at[cid, pl.ds(r0, ROWS_PER_SUB)])

    return sc_scatter


def _dinv_from(deg_ref):
    deg = deg_ref[0, :, 0] + deg_ref[1, :, 0] + 1.0
    return lax.rsqrt(deg)[:, None]


def _tc_scale(x, deg):
    def body(x_ref, deg_ref, o_ref):
        o_ref[...] = x_ref[...] * _dinv_from(deg_ref)

    return pl.pallas_call(
        body,
        grid=(NP // BLK,),
        in_specs=[
            pl.BlockSpec((BLK, 128), lambda i: (i, 0)),
            pl.BlockSpec((NC, BLK, DEG_W), lambda i: (0, i, 0)),
        ],
        out_specs=pl.BlockSpec((BLK, 128), lambda i: (i, 0)),
        out_shape=jax.ShapeDtypeStruct((NP, 128), jnp.float32),
    )(x, deg)


def _tc_mid(Z1, xs, deg, w1, b1):
    def body(z_ref, x_ref, deg_ref, w_ref, b_ref, o_ref):
        dinv = _dinv_from(deg_ref)
        u = dinv * (z_ref[0] + z_ref[1] + x_ref[...])
        h = jnp.dot(u, w_ref[...], preferred_element_type=jnp.float32)
        h = jnp.maximum(h + b_ref[...], 0.0)
        o_ref[...] = dinv * h

    return pl.pallas_call(
        body,
        grid=(NP // BLK,),
        in_specs=[
            pl.BlockSpec((NC, BLK, 128), lambda i: (0, i, 0)),
            pl.BlockSpec((BLK, 128), lambda i: (i, 0)),
            pl.BlockSpec((NC, BLK, DEG_W), lambda i: (0, i, 0)),
            pl.BlockSpec((128, 128), lambda i: (0, 0)),
            pl.BlockSpec((1, 128), lambda i: (0, 0)),
        ],
        out_specs=pl.BlockSpec((BLK, 128), lambda i: (i, 0)),
        out_shape=jax.ShapeDtypeStruct((NP, 128), jnp.float32),
    )(Z1, xs, deg, w1, b1)


def _tc_final(Z2, hs, deg, w2p, b2p):
    def body(z_ref, h_ref, deg_ref, w_ref, b_ref, o_ref):
        dinv = _dinv_from(deg_ref)
        u = dinv * (z_ref[0] + z_ref[1] + h_ref[...])
        o = jnp.dot(u, w_ref[...], preferred_element_type=jnp.float32)
        o = o + b_ref[...]
        col = lax.broadcasted_iota(jnp.int32, o.shape, 1)
        o_ref[...] = jnp.where(col >= 128, jnp.exp(o), o)

    return pl.pallas_call(
        body,
        grid=(NP // BLK,),
        in_specs=[
            pl.BlockSpec((NC, BLK, 128), lambda i: (0, i, 0)),
            pl.BlockSpec((BLK, 128), lambda i: (i, 0)),
            pl.BlockSpec((NC, BLK, DEG_W), lambda i: (0, i, 0)),
            pl.BlockSpec((128, W2P), lambda i: (0, 0)),
            pl.BlockSpec((1, W2P), lambda i: (0, 0)),
        ],
        out_specs=pl.BlockSpec((BLK, W2P), lambda i: (i, 0)),
        out_shape=jax.ShapeDtypeStruct((NP, W2P), jnp.float32),
    )(Z2, hs, deg, w2p, b2p)


def kernel(x, edge_index, W1, b1, W2, b2):
    n_edges = edge_index.shape[1]
    chunk = NW * CH
    n_edges_p = ((n_edges + chunk - 1) // chunk) * chunk

    src_p = jnp.pad(edge_index[0], (0, n_edges_p - n_edges),
                    constant_values=N).reshape(n_edges_p // CH, CH)
    dst_p = jnp.pad(edge_index[1], (0, n_edges_p - n_edges),
                    constant_values=N).reshape(n_edges_p // CH, CH)
    xp = jnp.pad(x, ((0, NP - N), (0, 0)))
    w2p = jnp.pad(W2, ((0, 0), (0, W2P - W2.shape[1])))
    b1r = b1.reshape(1, 128)
    b2p = jnp.pad(b2, (0, W2P - b2.shape[0])).reshape(1, W2P)

    sc_degree = _make_sc_degree(n_edges_p)
    sc_scatter = _make_sc_scatter(n_edges_p, chunks_c0=97)

    deg = sc_degree(dst_p)
    xs = _tc_scale(xp, deg)
    Z1 = sc_scatter(xs, src_p, dst_p)
    hs = _tc_mid(Z1, xs, deg, W1, b1r)
    Z2 = sc_scatter(hs, src_p, dst_p)
    out = _tc_final(Z2, hs, deg, w2p, b2p)

    pos = out[:N, :128]
    mass = out[:N, 128:129]
    return (pos, mass)


# speedup vs baseline: 1.5498x; 1.1011x over previous
import functools

import jax
import jax.numpy as jnp
from jax import lax
from jax.experimental import pallas as pl
from jax.experimental.pallas import tpu as pltpu
from jax.experimental.pallas import tpu_sc as plsc

NC, NS, LANES = 2, 16, 16
NW = NC * NS
CH = 128

N = 10000
NP = 10240
ROWS_PER_SUB = NP // NS
DEG_W = 16
BLK = 1280
W2P = 256

_MESH = plsc.VectorSubcoreMesh(core_axis_name="c", subcore_axis_name="s")
_SC_PARAMS = pltpu.CompilerParams(use_tc_tiling_on_sc=False)


def _zero_vmem(buf, nrows, width):
    lanes = LANES * 2 if buf.dtype == jnp.bfloat16 else LANES
    z = jnp.zeros((lanes,), buf.dtype)

    @pl.loop(0, nrows)
    def _(r):
        @pl.loop(0, width // lanes)
        def _(c):
            buf[r, pl.ds(c * lanes, lanes)] = z


def _make_sc_degree(n_edges_p):
    chunks_per_w = n_edges_p // (NW * CH)

    @functools.partial(
        pl.kernel,
        out_type=jax.ShapeDtypeStruct((NC, NP, DEG_W), jnp.float32),
        mesh=_MESH,
        compiler_params=_SC_PARAMS,
        scratch_types=[
            pltpu.VMEM((CH,), jnp.int32),
            pltpu.VMEM((CH, DEG_W), jnp.float32),
            pltpu.VMEM((CH, DEG_W), jnp.float32),
            pltpu.VMEM_SHARED((NP, DEG_W), jnp.float32),
        ],
    )
    def sc_degree(dst_hbm, out_hbm, idx_v, ones_v, zbuf, acc):
        cid = lax.axis_index("c")
        sid = lax.axis_index("s")
        wid = cid * NS + sid
        base = wid * chunks_per_w
        one = jnp.full((LANES,), 1.0, jnp.float32)

        @pl.loop(0, CH)
        def _(r):
            ones_v[r, :] = one

        _zero_vmem(zbuf, CH, DEG_W)

        @pl.loop(0, ROWS_PER_SUB // CH)
        def _(j):
            pltpu.sync_copy(zbuf, acc.at[pl.ds(sid * ROWS_PER_SUB + j * CH, CH)])

        plsc.subcore_barrier()

        @pl.loop(0, chunks_per_w)
        def _(i):
            pltpu.sync_copy(dst_hbm.at[base + i], idx_v)
            pltpu.sync_copy(ones_v, acc.at[idx_v], add=True)

        plsc.subcore_barrier()
        r0 = sid * ROWS_PER_SUB
        pltpu.sync_copy(acc.at[pl.ds(r0, ROWS_PER_SUB)],
                        out_hbm.at[cid, pl.ds(r0, ROWS_PER_SUB)])

    return sc_degree


def _make_sc_scatter(n_edges_p, chunks_c0, dtype):
    total_chunks = n_edges_p // CH
    chunks_c1 = total_chunks // NS - chunks_c0

    @functools.partial(
        pl.kernel,
        out_type=jax.ShapeDtypeStruct((NC, NP, 128), dtype),
        mesh=_MESH,
        compiler_params=_SC_PARAMS,
        scratch_types=[
            pltpu.VMEM((CH,), jnp.int32),
            pltpu.VMEM((CH,), jnp.int32),
            pltpu.VMEM((CH, 128), dtype),
            pltpu.VMEM_SHARED((NP, 128), dtype),
            pltpu.SemaphoreType.DMA,
        ],
    )
    def sc_scatter(y_hbm, src_hbm, dst_hbm, out_hbm, srcv, dstv, rows, acc, sem):
        cid = lax.axis_index("c")
        sid = lax.axis_index("s")
        nch = jnp.where(cid == 0, chunks_c0, chunks_c1)
        base = cid * NS * chunks_c0 + sid * nch

        _zero_vmem(rows, CH, 128)

        @pl.loop(0, ROWS_PER_SUB // CH)
        def _(j):
            pltpu.sync_copy(rows, acc.at[pl.ds(sid * ROWS_PER_SUB + j * CH, CH)])

        plsc.subcore_barrier()

        @pl.loop(0, nch)
        def _(i):
            pltpu.sync_copy(src_hbm.at[base + i], srcv)
            pltpu.sync_copy(dst_hbm.at[base + i], dstv)
            pltpu.async_copy(y_hbm.at[srcv], rows, sem).wait()
            pltpu.sync_copy(rows, acc.at[dstv], add=True)

        plsc.subcore_barrier()
        r0 = sid * ROWS_PER_SUB
        pltpu.sync_copy(acc.at[pl.ds(r0, ROWS_PER_SUB)],
                        out_hbm.at[cid, pl.ds(r0, ROWS_PER_SUB)])

    return sc_scatter


def _dinv_from(deg_ref):
    deg = deg_ref[0, :, 0] + deg_ref[1, :, 0] + 1.0
    return lax.rsqrt(deg)[:, None]


def _tc_scale(x, deg):
    def body(x_ref, deg_ref, o_ref):
        o_ref[...] = (x_ref[...] * _dinv_from(deg_ref)).astype(jnp.bfloat16)

    return pl.pallas_call(
        body,
        grid=(NP // BLK,),
        in_specs=[
            pl.BlockSpec((BLK, 128), lambda i: (i, 0)),
            pl.BlockSpec((NC, BLK, DEG_W), lambda i: (0, i, 0)),
        ],
        out_specs=pl.BlockSpec((BLK, 128), lambda i: (i, 0)),
        out_shape=jax.ShapeDtypeStruct((NP, 128), jnp.bfloat16),
    )(x, deg)


def _tc_mid(Z1, xs, deg, w1, b1):
    def body(z_ref, x_ref, deg_ref, w_ref, b_ref, o_ref):
        dinv = _dinv_from(deg_ref)
        zsum = (z_ref[0].astype(jnp.float32) + z_ref[1].astype(jnp.float32)
                + x_ref[...].astype(jnp.float32))
        u = dinv * zsum
        h = jnp.dot(u, w_ref[...], preferred_element_type=jnp.float32)
        h = jnp.maximum(h + b_ref[...], 0.0)
        o_ref[...] = dinv * h

    return pl.pallas_call(
        body,
        grid=(NP // BLK,),
        in_specs=[
            pl.BlockSpec((NC, BLK, 128), lambda i: (0, i, 0)),
            pl.BlockSpec((BLK, 128), lambda i: (i, 0)),
            pl.BlockSpec((NC, BLK, DEG_W), lambda i: (0, i, 0)),
            pl.BlockSpec((128, 128), lambda i: (0, 0)),
            pl.BlockSpec((1, 128), lambda i: (0, 0)),
        ],
        out_specs=pl.BlockSpec((BLK, 128), lambda i: (i, 0)),
        out_shape=jax.ShapeDtypeStruct((NP, 128), jnp.float32),
    )(Z1, xs, deg, w1, b1)


def _tc_final(Z2, hs, deg, w2p, b2p):
    def body(z_ref, h_ref, deg_ref, w_ref, b_ref, o_ref):
        dinv = _dinv_from(deg_ref)
        u = dinv * (z_ref[0] + z_ref[1] + h_ref[...])
        o = jnp.dot(u, w_ref[...], preferred_element_type=jnp.float32)
        o = o + b_ref[...]
        col = lax.broadcasted_iota(jnp.int32, o.shape, 1)
        o_ref[...] = jnp.where(col >= 128, jnp.exp(o), o)

    return pl.pallas_call(
        body,
        grid=(NP // BLK,),
        in_specs=[
            pl.BlockSpec((NC, BLK, 128), lambda i: (0, i, 0)),
            pl.BlockSpec((BLK, 128), lambda i: (i, 0)),
            pl.BlockSpec((NC, BLK, DEG_W), lambda i: (0, i, 0)),
            pl.BlockSpec((128, W2P), lambda i: (0, 0)),
            pl.BlockSpec((1, W2P), lambda i: (0, 0)),
        ],
        out_specs=pl.BlockSpec((BLK, W2P), lambda i: (i, 0)),
        out_shape=jax.ShapeDtypeStruct((NP, W2P), jnp.float32),
    )(Z2, hs, deg, w2p, b2p)


def kernel(x, edge_index, W1, b1, W2, b2):
    n_edges = edge_index.shape[1]
    chunk = NW * CH
    n_edges_p = ((n_edges + chunk - 1) // chunk) * chunk

    src_p = jnp.pad(edge_index[0], (0, n_edges_p - n_edges),
                    constant_values=N).reshape(n_edges_p // CH, CH)
    dst_p = jnp.pad(edge_index[1], (0, n_edges_p - n_edges),
                    constant_values=N).reshape(n_edges_p // CH, CH)
    xp = jnp.pad(x, ((0, NP - N), (0, 0)))
    w2p = jnp.pad(W2, ((0, 0), (0, W2P - W2.shape[1])))
    b1r = b1.reshape(1, 128)
    b2p = jnp.pad(b2, (0, W2P - b2.shape[0])).reshape(1, W2P)

    sc_degree = _make_sc_degree(n_edges_p)
    sc_scatter_bf16 = _make_sc_scatter(n_edges_p, 97, jnp.bfloat16)
    sc_scatter_f32 = _make_sc_scatter(n_edges_p, 97, jnp.float32)

    deg = sc_degree(dst_p)
    xs = _tc_scale(xp, deg)
    Z1 = sc_scatter_bf16(xs, src_p, dst_p)
    hs = _tc_mid(Z1, xs, deg, W1, b1r)
    Z2 = sc_scatter_f32(hs, src_p, dst_p)
    out = _tc_final(Z2, hs, deg, w2p, b2p)

    pos = out[:N, :128]
    mass = out[:N, 128:129]
    return (pos, mass)


# speedup vs baseline: 1.7394x; 1.1224x over previous
import functools

import jax
import jax.numpy as jnp
from jax import lax
from jax.experimental import pallas as pl
from jax.experimental.pallas import tpu as pltpu
from jax.experimental.pallas import tpu_sc as plsc

NC, NS, LANES = 2, 16, 16
NW = NC * NS
CH = 128

N = 10000
NP = 10240
ROWS_PER_SUB = NP // NS
DEG_W = 16
BLK = 1280
W2P = 256

_MESH = plsc.VectorSubcoreMesh(core_axis_name="c", subcore_axis_name="s")
_SC_PARAMS = pltpu.CompilerParams(use_tc_tiling_on_sc=False)


def _zero_vmem(buf, nrows, width):
    lanes = LANES * 2 if buf.dtype == jnp.bfloat16 else LANES
    z = jnp.zeros((lanes,), buf.dtype)

    @pl.loop(0, nrows)
    def _(r):
        @pl.loop(0, width // lanes)
        def _(c):
            buf[r, pl.ds(c * lanes, lanes)] = z


def _make_sc_degree(n_edges_p):
    chunks_per_w = n_edges_p // (NW * CH)

    @functools.partial(
        pl.kernel,
        out_type=jax.ShapeDtypeStruct((NC, NP, DEG_W), jnp.float32),
        mesh=_MESH,
        compiler_params=_SC_PARAMS,
        scratch_types=[
            pltpu.VMEM((CH,), jnp.int32),
            pltpu.VMEM((CH, DEG_W), jnp.float32),
            pltpu.VMEM((CH, DEG_W), jnp.float32),
            pltpu.VMEM_SHARED((NP, DEG_W), jnp.float32),
        ],
    )
    def sc_degree(dst_hbm, out_hbm, idx_v, ones_v, zbuf, acc):
        cid = lax.axis_index("c")
        sid = lax.axis_index("s")
        wid = cid * NS + sid
        base = wid * chunks_per_w
        one = jnp.full((LANES,), 1.0, jnp.float32)

        @pl.loop(0, CH)
        def _(r):
            ones_v[r, :] = one

        _zero_vmem(zbuf, CH, DEG_W)

        @pl.loop(0, ROWS_PER_SUB // CH)
        def _(j):
            pltpu.sync_copy(zbuf, acc.at[pl.ds(sid * ROWS_PER_SUB + j * CH, CH)])

        plsc.subcore_barrier()

        @pl.loop(0, chunks_per_w)
        def _(i):
            pltpu.sync_copy(dst_hbm.at[base + i], idx_v)
            pltpu.sync_copy(ones_v, acc.at[idx_v], add=True)

        plsc.subcore_barrier()
        r0 = sid * ROWS_PER_SUB
        pltpu.sync_copy(acc.at[pl.ds(r0, ROWS_PER_SUB)],
                        out_hbm.at[cid, pl.ds(r0, ROWS_PER_SUB)])

    return sc_degree


def _make_sc_scatter(n_edges_p, chunks_c0, dtype):
    total_chunks = n_edges_p // CH
    chunks_c1 = total_chunks // NS - chunks_c0

    @functools.partial(
        pl.kernel,
        out_type=jax.ShapeDtypeStruct((NC, NP, 128), dtype),
        mesh=_MESH,
        compiler_params=_SC_PARAMS,
        scratch_types=[
            pltpu.VMEM((CH,), jnp.int32),
            pltpu.VMEM((CH,), jnp.int32),
            pltpu.VMEM((CH, 128), dtype),
            pltpu.VMEM_SHARED((NP, 128), dtype),
            pltpu.SemaphoreType.DMA,
        ],
    )
    def sc_scatter(y_hbm, src_hbm, dst_hbm, out_hbm, srcv, dstv, rows, acc, sem):
        cid = lax.axis_index("c")
        sid = lax.axis_index("s")
        nch = jnp.where(cid == 0, chunks_c0, chunks_c1)
        base = cid * NS * chunks_c0 + sid * nch

        _zero_vmem(rows, CH, 128)

        @pl.loop(0, ROWS_PER_SUB // CH)
        def _(j):
            pltpu.sync_copy(rows, acc.at[pl.ds(sid * ROWS_PER_SUB + j * CH, CH)])

        plsc.subcore_barrier()

        @pl.loop(0, nch)
        def _(i):
            pltpu.sync_copy(src_hbm.at[base + i], srcv)
            pltpu.sync_copy(dst_hbm.at[base + i], dstv)
            pltpu.async_copy(y_hbm.at[srcv], rows, sem).wait()
            pltpu.sync_copy(rows, acc.at[dstv], add=True)

        plsc.subcore_barrier()
        r0 = sid * ROWS_PER_SUB
        pltpu.sync_copy(acc.at[pl.ds(r0, ROWS_PER_SUB)],
                        out_hbm.at[cid, pl.ds(r0, ROWS_PER_SUB)])

    return sc_scatter


def _dinv_from(deg_ref):
    deg = deg_ref[0, :, 0] + deg_ref[1, :, 0] + 1.0
    return lax.rsqrt(deg)[:, None]


def _tc_scale(x, deg):
    def body(x_ref, deg_ref, o_ref):
        o_ref[...] = (x_ref[...] * _dinv_from(deg_ref)).astype(jnp.bfloat16)

    return pl.pallas_call(
        body,
        grid=(NP // BLK,),
        in_specs=[
            pl.BlockSpec((BLK, 128), lambda i: (i, 0)),
            pl.BlockSpec((NC, BLK, DEG_W), lambda i: (0, i, 0)),
        ],
        out_specs=pl.BlockSpec((BLK, 128), lambda i: (i, 0)),
        out_shape=jax.ShapeDtypeStruct((NP, 128), jnp.bfloat16),
    )(x, deg)


def _tc_mid(Z1, xs, deg, w1, b1):
    def body(z_ref, x_ref, deg_ref, w_ref, b_ref, o_ref):
        dinv = _dinv_from(deg_ref)
        zsum = (z_ref[0].astype(jnp.float32) + z_ref[1].astype(jnp.float32)
                + x_ref[...].astype(jnp.float32))
        u = dinv * zsum
        h = jnp.dot(u, w_ref[...], preferred_element_type=jnp.float32)
        h = jnp.maximum(h + b_ref[...], 0.0)
        o_ref[...] = (dinv * h).astype(o_ref.dtype)

    return pl.pallas_call(
        body,
        grid=(NP // BLK,),
        in_specs=[
            pl.BlockSpec((NC, BLK, 128), lambda i: (0, i, 0)),
            pl.BlockSpec((BLK, 128), lambda i: (i, 0)),
            pl.BlockSpec((NC, BLK, DEG_W), lambda i: (0, i, 0)),
            pl.BlockSpec((128, 128), lambda i: (0, 0)),
            pl.BlockSpec((1, 128), lambda i: (0, 0)),
        ],
        out_specs=pl.BlockSpec((BLK, 128), lambda i: (i, 0)),
        out_shape=jax.ShapeDtypeStruct((NP, 128), jnp.bfloat16),
    )(Z1, xs, deg, w1, b1)


def _tc_final(Z2, hs, deg, w2p, b2p):
    def body(z_ref, h_ref, deg_ref, w_ref, b_ref, o_ref):
        dinv = _dinv_from(deg_ref)
        u = dinv * (z_ref[0].astype(jnp.float32) + z_ref[1].astype(jnp.float32)
                    + h_ref[...].astype(jnp.float32))
        o = jnp.dot(u, w_ref[...], preferred_element_type=jnp.float32)
        o = o + b_ref[...]
        col = lax.broadcasted_iota(jnp.int32, o.shape, 1)
        o_ref[...] = jnp.where(col >= 128, jnp.exp(o), o)

    return pl.pallas_call(
        body,
        grid=(NP // BLK,),
        in_specs=[
            pl.BlockSpec((NC, BLK, 128), lambda i: (0, i, 0)),
            pl.BlockSpec((BLK, 128), lambda i: (i, 0)),
            pl.BlockSpec((NC, BLK, DEG_W), lambda i: (0, i, 0)),
            pl.BlockSpec((128, W2P), lambda i: (0, 0)),
            pl.BlockSpec((1, W2P), lambda i: (0, 0)),
        ],
        out_specs=pl.BlockSpec((BLK, W2P), lambda i: (i, 0)),
        out_shape=jax.ShapeDtypeStruct((NP, W2P), jnp.float32),
    )(Z2, hs, deg, w2p, b2p)


def kernel(x, edge_index, W1, b1, W2, b2):
    n_edges = edge_index.shape[1]
    chunk = NW * CH
    n_edges_p = ((n_edges + chunk - 1) // chunk) * chunk

    src_p = jnp.pad(edge_index[0], (0, n_edges_p - n_edges),
                    constant_values=N).reshape(n_edges_p // CH, CH)
    dst_p = jnp.pad(edge_index[1], (0, n_edges_p - n_edges),
                    constant_values=N).reshape(n_edges_p // CH, CH)
    xp = jnp.pad(x, ((0, NP - N), (0, 0)))
    w2p = jnp.pad(W2, ((0, 0), (0, W2P - W2.shape[1])))
    b1r = b1.reshape(1, 128)
    b2p = jnp.pad(b2, (0, W2P - b2.shape[0])).reshape(1, W2P)

    sc_degree = _make_sc_degree(n_edges_p)
    sc_scatter_bf16 = _make_sc_scatter(n_edges_p, 97, jnp.bfloat16)
    sc_scatter_f32 = _make_sc_scatter(n_edges_p, 97, jnp.float32)

    deg = sc_degree(dst_p)
    xs = _tc_scale(xp, deg)
    Z1 = sc_scatter_bf16(xs, src_p, dst_p)
    hs = _tc_mid(Z1, xs, deg, W1, b1r)
    Z2 = sc_scatter_bf16(hs, src_p, dst_p)
    out = _tc_final(Z2, hs, deg, w2p, b2p)

    pos = out[:N, :128]
    mass = out[:N, 128:129]
    return (pos, mass)
